# Initial kernel scaffold; baseline (speedup 1.0000x reference)
import jax
import jax.numpy as jnp
from jax.experimental import pallas as pl


def kernel(x, indices, weight, bias, attention_weights):
    raise NotImplementedError("write your pallas kernel here")


# speedup vs baseline: 1.4681x; 1.4681x over previous
import functools

import jax
import jax.numpy as jnp
from jax import lax
from jax.experimental import pallas as pl
from jax.experimental.pallas import tpu as pltpu
from jax.experimental.pallas import tpu_sc as plsc

_NC = 2
_NS = 16
_NW = _NC * _NS
_VEC = 16
_PACK = 128


def _sc_gather(weight, ab_pack, indices):
    k_short = indices.shape[0]
    d = weight.shape[1]
    vals = _PACK // 4
    rows_per_w = k_short // _NW
    chunk = min(rows_per_w, 256)
    mesh = plsc.VectorSubcoreMesh(core_axis_name="c", subcore_axis_name="s")
    kvec = jax.ShapeDtypeStruct((k_short,), jnp.float32)

    @functools.partial(
        pl.kernel,
        mesh=mesh,
        compiler_params=pltpu.CompilerParams(needs_layout_passes=False),
        out_type=[jax.ShapeDtypeStruct((k_short, d), jnp.float32),
                  kvec, kvec, kvec, kvec],
        scratch_types=[
            pltpu.VMEM((rows_per_w,), jnp.int32),
            pltpu.VMEM((rows_per_w,), jnp.int32),
            pltpu.VMEM((rows_per_w,), jnp.int32),
            pltpu.VMEM((chunk, d), jnp.float32),
            pltpu.VMEM((chunk, _PACK), jnp.float32),
            pltpu.VMEM((chunk,), jnp.float32),
            pltpu.VMEM((chunk,), jnp.float32),
            pltpu.VMEM((chunk,), jnp.float32),
            pltpu.VMEM((chunk,), jnp.float32),
            pltpu.SemaphoreType.DMA,
            pltpu.SemaphoreType.DMA,
        ],
    )
    def gather_kernel(w_hbm, pack_hbm, idx_hbm, w_out, a0_out, a1_out,
                      a2_out, b_out, idx_v, prow_v, loff_v, rows_v, pack_v,
                      s0_v, s1_v, s2_v, s3_v, sem, out_sem):
        wid = lax.axis_index("s") * _NC + lax.axis_index("c")
        base = wid * rows_per_w
        pltpu.sync_copy(idx_hbm.at[pl.ds(base, rows_per_w)], idx_v)

        @pl.loop(0, rows_per_w, step=_VEC)
        def _(o):
            v = idx_v[pl.ds(o, _VEC)]
            prow_v[pl.ds(o, _VEC)] = lax.shift_right_logical(v, 5)
            loff_v[pl.ds(o, _VEC)] = lax.shift_left(v & 31, 2)

        @pl.loop(0, rows_per_w, step=chunk)
        def _(cc):
            cp_w = pltpu.async_copy(w_hbm.at[idx_v.at[pl.ds(cc, chunk)]],
                                    rows_v, sem)
            cp_p = pltpu.async_copy(pack_hbm.at[prow_v.at[pl.ds(cc, chunk)]],
                                    pack_v, sem)
            cp_w.wait()
            cp_p.wait()

            @pl.loop(0, chunk, step=_VEC)
            def _(g):
                rid = lax.iota(jnp.int32, _VEC) + g
                lo = loff_v[pl.ds(cc + g, _VEC)]
                s0_v[pl.ds(g, _VEC)] = plsc.load_gather(pack_v, [rid, lo])
                s1_v[pl.ds(g, _VEC)] = plsc.load_gather(pack_v, [rid, lo + 1])
                s2_v[pl.ds(g, _VEC)] = plsc.load_gather(pack_v, [rid, lo + 2])
                s3_v[pl.ds(g, _VEC)] = plsc.load_gather(pack_v, [rid, lo + 3])

            dst = pl.ds(base + cc, chunk)
            cp0 = pltpu.async_copy(s0_v, a0_out.at[dst], out_sem)
            cp1 = pltpu.async_copy(s1_v, a1_out.at[dst], out_sem)
            cp2 = pltpu.async_copy(s2_v, a2_out.at[dst], out_sem)
            cp3 = pltpu.async_copy(s3_v, b_out.at[dst], out_sem)
            pltpu.sync_copy(rows_v, w_out.at[dst])
            cp0.wait()
            cp1.wait()
            cp2.wait()
            cp3.wait()

    return gather_kernel(weight, ab_pack, indices)


def _tc_body(n_j, d, x_ref, w_ref, a0_ref, a1_ref, a2_ref, b_ref, o_ref):
    l0, l1, l2 = a0_ref[...], a1_ref[...], a2_ref[...]
    m = jnp.maximum(jnp.maximum(l0, l1), l2)
    e0, e1, e2 = jnp.exp(l0 - m), jnp.exp(l1 - m), jnp.exp(l2 - m)
    inv = 1.0 / (e0 + e1 + e2)
    a = (e0 * inv, e1 * inv, e2 * inv)
    w_bf = w_ref[...].astype(jnp.bfloat16)
    acc = jnp.broadcast_to(b_ref[...], o_ref.shape)
    for j in range(n_j):
        xj = x_ref[:, j * d:(j + 1) * d]
        g = lax.dot_general(xj, w_bf, (((1,), (1,)), ((), ())),
                            preferred_element_type=jnp.float32)
        acc = acc + g * a[j]
    o_ref[...] = acc


def _tc_matmul(x, w_g, a0, a1, a2, b, kblk=2048):
    bsz, three_i = x.shape
    k_short, d = w_g.shape
    n_j = three_i // d
    row_spec = pl.BlockSpec((1, kblk), lambda i: (0, i))

    return pl.pallas_call(
        functools.partial(_tc_body, n_j, d),
        grid=(k_short // kblk,),
        in_specs=[
            pl.BlockSpec((bsz, three_i), lambda i: (0, 0)),
            pl.BlockSpec((kblk, d), lambda i: (i, 0)),
            row_spec, row_spec, row_spec, row_spec,
        ],
        out_specs=pl.BlockSpec((bsz, kblk), lambda i: (0, i)),
        out_shape=jax.ShapeDtypeStruct((bsz, k_short), jnp.float32),
    )(x, w_g, a0, a1, a2, b)


def kernel(x, indices, weight, bias, attention_weights):
    n_att = attention_weights.shape[1]
    ab4 = jnp.concatenate([attention_weights, bias[:, None]], axis=1)
    ab_pack = ab4.reshape(-1, _PACK)
    w_g, a0, a1, a2, b = _sc_gather(weight, ab_pack, indices)
    assert n_att == 3
    k_short = indices.shape[0]
    return _tc_matmul(x.astype(jnp.bfloat16), w_g,
                      a0.reshape(1, k_short), a1.reshape(1, k_short),
                      a2.reshape(1, k_short), b.reshape(1, k_short))


# speedup vs baseline: 1.9323x; 1.3161x over previous
import functools

import jax
import jax.numpy as jnp
from jax import lax
from jax.experimental import pallas as pl
from jax.experimental.pallas import tpu as pltpu
from jax.experimental.pallas import tpu_sc as plsc

_NC = 2
_NS = 16
_NW = _NC * _NS


def _sc_gather(weight, indices):
    k_short = indices.shape[0]
    d = weight.shape[1]
    rows_per_w = k_short // _NW
    chunk = min(rows_per_w, 128)
    mesh = plsc.VectorSubcoreMesh(core_axis_name="c", subcore_axis_name="s")

    @functools.partial(
        pl.kernel,
        mesh=mesh,
        out_type=jax.ShapeDtypeStruct((k_short, d), jnp.float32),
        scratch_types=[
            pltpu.VMEM((rows_per_w,), jnp.int32),
            pltpu.VMEM((chunk, d), jnp.float32),
            pltpu.VMEM((chunk, d), jnp.float32),
            pltpu.SemaphoreType.DMA,
            pltpu.SemaphoreType.DMA,
            pltpu.SemaphoreType.DMA,
            pltpu.SemaphoreType.DMA,
        ],
    )
    def gather_kernel(w_hbm, idx_hbm, w_out, idx_v, rows_v0, rows_v1,
                      gsem0, gsem1, osem0, osem1):
        rows_b = (rows_v0, rows_v1)
        gsem = (gsem0, gsem1)
        osem = (osem0, osem1)
        wid = lax.axis_index("s") * _NC + lax.axis_index("c")
        base = wid * rows_per_w
        pltpu.sync_copy(idx_hbm.at[pl.ds(base, rows_per_w)], idx_v)

        n_chunks = rows_per_w // chunk

        def fire_gather(c, b):
            return pltpu.async_copy(
                w_hbm.at[idx_v.at[pl.ds(c * chunk, chunk)]], rows_b[b],
                gsem[b])

        def fire_out(c, b):
            return pltpu.async_copy(
                rows_b[b], w_out.at[pl.ds(base + c * chunk, chunk)], osem[b])

        pending_g = {0: fire_gather(0, 0)}
        pending_o = {}
        for c in range(n_chunks):
            b = c & 1
            if c + 1 < n_chunks:
                if c >= 1:
                    pending_o.pop(c - 1).wait()
                pending_g[c + 1] = fire_gather(c + 1, 1 - b)
            pending_g.pop(c).wait()
            pending_o[c] = fire_out(c, b)
        for c in sorted(pending_o):
            pending_o.pop(c).wait()

    return gather_kernel(weight, indices)


def _tc_body(n_j, d, x_ref, w_ref, at_ref, b_ref, o_ref):
    att = at_ref[...]
    m = jnp.max(att, axis=0, keepdims=True)
    e = jnp.exp(att - m)
    a = e / jnp.sum(e, axis=0, keepdims=True)
    w_bf = w_ref[...].astype(jnp.bfloat16)
    acc = jnp.broadcast_to(b_ref[...], o_ref.shape)
    for j in range(n_j):
        xj = x_ref[:, j * d:(j + 1) * d]
        g = lax.dot_general(xj, w_bf, (((1,), (1,)), ((), ())),
                            preferred_element_type=jnp.float32)
        acc = acc + g * a[j:j + 1, :]
    o_ref[...] = acc


def _tc_matmul(x, w_g, att_t, bias_r, kblk=2048):
    bsz, three_i = x.shape
    k_short, d = w_g.shape
    n_j = three_i // d

    return pl.pallas_call(
        functools.partial(_tc_body, n_j, d),
        grid=(k_short // kblk,),
        in_specs=[
            pl.BlockSpec((bsz, three_i), lambda i: (0, 0)),
            pl.BlockSpec((kblk, d), lambda i: (i, 0)),
            pl.BlockSpec((n_j, kblk), lambda i: (0, i)),
            pl.BlockSpec((1, kblk), lambda i: (0, i)),
        ],
        out_specs=pl.BlockSpec((bsz, kblk), lambda i: (0, i)),
        out_shape=jax.ShapeDtypeStruct((bsz, k_short), jnp.float32),
    )(x, w_g, att_t, bias_r)


def kernel(x, indices, weight, bias, attention_weights):
    k_short = indices.shape[0]
    w_g = _sc_gather(weight, indices)
    att_t = jnp.take(attention_weights, indices, axis=0).T
    bias_r = jnp.take(bias, indices).reshape(1, k_short)
    return _tc_matmul(x.astype(jnp.bfloat16), w_g, att_t, bias_r)
